# Initial kernel scaffold; baseline (speedup 1.0000x reference)
import jax
import jax.numpy as jnp
from jax.experimental import pallas as pl


def kernel(x, edge_index, triples, step_ids, W1, b1, W2, b2, Wd1, bd1, Wd2, bd2):
    raise NotImplementedError("write your pallas kernel here")


# speedup vs baseline: 2.8047x; 2.8047x over previous
import jax
import jax.numpy as jnp
from jax import lax
from jax.experimental import pallas as pl
from jax.experimental.pallas import tpu as pltpu
from jax.experimental.pallas import tpu_sc as plsc

N, D, E, T, S = 10000, 128, 320000, 100000, 4
NC, NS = 2, 16
NW = NC * NS
SINK = N
NPAD = 10240
RB = 512
NBLK = NPAD // RB
RSUB = NPAD // NS

EC = 10
EKI = 8
E_PAD = NW * EC * EKI * 128

TCH = 32
T_PAD = NW * TCH * 128

_mesh = plsc.VectorSubcoreMesh(
    core_axis_name="c", subcore_axis_name="s", num_cores=NC, num_subcores=NS)


def _edge_agg_body(h_hbm, src_hbm, dst_hbm, parts_out,
                   sidx, didx, didx1, rows, sem0, sem1, acc_sh):
  sems = (sem0, sem1)
  core = lax.axis_index("c")
  sub = lax.axis_index("s")
  wid = sub * NC + core
  r0 = sub * RSUB

  zv = jnp.zeros((16,), jnp.float32)

  def zrow(i, carry):
    for j in range(D // 16):
      rows[i, pl.ds(16 * j, 16)] = zv
    return carry

  lax.fori_loop(0, 128, zrow, 0)
  for m in range(RSUB // 128):
    pltpu.sync_copy(rows.at[pl.ds(0, 128)],
                    acc_sh.at[pl.ds(r0 + m * 128, 128)])
  plsc.subcore_barrier()

  def chunk(k, carry):
    base = (wid * EC + k) * EKI
    pltpu.sync_copy(src_hbm.at[pl.ds(base, EKI)], sidx)
    pltpu.sync_copy(dst_hbm.at[pl.ds(base, EKI)], didx)
    descs = [None, None]
    descs[0] = pltpu.async_copy(h_hbm.at[sidx.at[0]],
                                rows.at[pl.ds(0, 128)], sems[0])
    for j in range(EKI):
      if j + 1 < EKI:
        slot = (j + 1) % 2
        descs[slot] = pltpu.async_copy(h_hbm.at[sidx.at[j + 1]],
                                       rows.at[pl.ds(slot * 128, 128)],
                                       sems[slot])
      for q in range(8):
        sl = pl.ds(16 * q, 16)
        didx1[sl] = didx[j, sl]
      descs[j % 2].wait()
      pltpu.sync_copy(rows.at[pl.ds((j % 2) * 128, 128)],
                      acc_sh.at[didx1], add=True)
    return carry

  lax.fori_loop(0, EC, chunk, 0)
  plsc.subcore_barrier()
  for m in range(RSUB // 128):
    pltpu.sync_copy(acc_sh.at[pl.ds(r0 + m * 128, 128)],
                    rows.at[pl.ds(0, 128)])
    pltpu.sync_copy(rows.at[pl.ds(0, 128)],
                    parts_out.at[core, pl.ds(r0 + m * 128, 128)])


def _edge_agg(h, src2d, dst2d):
  return pl.kernel(
      _edge_agg_body,
      out_type=jax.ShapeDtypeStruct((NC, NPAD, D), jnp.float32),
      mesh=_mesh,
      scratch_types=[
          pltpu.VMEM((EKI, 128), jnp.int32),
          pltpu.VMEM((EKI, 128), jnp.int32),
          pltpu.VMEM((128,), jnp.int32),
          pltpu.VMEM((256, D), jnp.float32),
          pltpu.SemaphoreType.DMA,
          pltpu.SemaphoreType.DMA,
          pltpu.VMEM_SHARED((NPAD, D), jnp.float32),
      ],
  )(h, src2d, dst2d)


def _deg_body(dst_hbm, ones_hbm, z128_hbm, deg_out, didx, didx1, ones_v,
              deg_sh):
  core = lax.axis_index("c")
  sub = lax.axis_index("s")
  wid = sub * NC + core
  r0 = sub * RSUB
  pltpu.sync_copy(z128_hbm.at[pl.ds(r0, RSUB)], deg_sh.at[pl.ds(r0, RSUB)])
  pltpu.sync_copy(ones_hbm, ones_v)
  plsc.subcore_barrier()

  def chunk(k, carry):
    base = (wid * EC + k) * EKI
    pltpu.sync_copy(dst_hbm.at[pl.ds(base, EKI)], didx)
    for j in range(EKI):
      for q in range(8):
        sl = pl.ds(16 * q, 16)
        didx1[sl] = didx[j, sl]
      pltpu.sync_copy(ones_v, deg_sh.at[didx1], add=True)
    return carry

  lax.fori_loop(0, EC, chunk, 0)
  plsc.subcore_barrier()
  pltpu.sync_copy(deg_sh.at[pl.ds(r0, RSUB)],
                  deg_out.at[core, pl.ds(r0, RSUB)])


def _deg_sc(dst2d, ones128, zacc):
  return pl.kernel(
      _deg_body,
      out_type=jax.ShapeDtypeStruct((NC, NPAD, D), jnp.float32),
      mesh=_mesh,
      scratch_types=[
          pltpu.VMEM((EKI, 128), jnp.int32),
          pltpu.VMEM((128,), jnp.int32),
          pltpu.VMEM((128, D), jnp.float32),
          pltpu.VMEM_SHARED((NPAD, D), jnp.float32),
      ],
  )(dst2d, ones128, zacc)


def _layer_body(p_ref, dg_ref, w_ref, b_ref, a_ref, bm_ref, c_ref, bd_ref,
                h_ref, pa_ref, pb_ref, pc_ref):
  p = p_ref[0] + p_ref[1]
  dg = dg_ref[0, :, 0:1] + dg_ref[1, :, 0:1]
  agg = p * (1.0 / jnp.maximum(dg, 1.0))
  h = jnp.maximum(
      jnp.dot(agg, w_ref[...], preferred_element_type=jnp.float32) + b_ref[0],
      0.0)
  h_ref[...] = h
  pa_ref[...] = jnp.dot(h, a_ref[...],
                        preferred_element_type=jnp.float32) + bd_ref[0]
  pb_ref[...] = jnp.dot(h, bm_ref[...], preferred_element_type=jnp.float32)
  pc_ref[...] = jnp.dot(h, c_ref[...], preferred_element_type=jnp.float32)


def _layer_tc(parts, degparts, W, b, A, Bm, C, bd1):
  full = lambda shape: pl.BlockSpec(shape, lambda i: (0,) * len(shape))
  return pl.pallas_call(
      _layer_body,
      grid=(NBLK,),
      in_specs=[
          pl.BlockSpec((NC, RB, D), lambda i: (0, i, 0)),
          pl.BlockSpec((NC, RB, D), lambda i: (0, i, 0)),
          full((D, D)), full((1, D)),
          full((D, D)), full((D, D)), full((D, D)), full((1, D)),
      ],
      out_specs=[pl.BlockSpec((RB, D), lambda i: (i, 0))] * 4,
      out_shape=[jax.ShapeDtypeStruct((NPAD, D), jnp.float32)] * 4,
  )(parts, degparts, W, b.reshape(1, D), A, Bm, C, bd1.reshape(1, D))


def _decoder_body(pa_hbm, pb_hbm, pc_hbm, pl_hbm, sn_hbm, dn_hbm, st_hbm,
                  wd2_hbm, out_hbm,
                  pslab, snslab, dnslab, stslab, ia, ib, ic,
                  rowsA, rowsB, rowsC, o16, wd2v, sem):
  core = lax.axis_index("c")
  sub = lax.axis_index("s")
  wid = sub * NC + core
  slab0 = wid * TCH
  pltpu.sync_copy(wd2_hbm, wd2v)
  pltpu.sync_copy(pl_hbm.at[pl.ds(slab0, TCH)], pslab)
  pltpu.sync_copy(sn_hbm.at[pl.ds(slab0, TCH)], snslab)
  pltpu.sync_copy(dn_hbm.at[pl.ds(slab0, TCH)], dnslab)
  pltpu.sync_copy(st_hbm.at[pl.ds(slab0, TCH)], stslab)
  wregs = [wd2v[pl.ds(16 * j, 16)] for j in range(8)]

  def chunk(k, carry):
    for j in range(8):
      sl = pl.ds(16 * j, 16)
      base = stslab[k, sl] * NPAD
      ia[sl] = base + pslab[k, sl]
      ib[sl] = base + snslab[k, sl]
      ic[sl] = base + dnslab[k, sl]
    d1 = pltpu.async_copy(pa_hbm.at[ia], rowsA, sem)
    d2 = pltpu.async_copy(pb_hbm.at[ib], rowsB, sem)
    d3 = pltpu.async_copy(pc_hbm.at[ic], rowsC, sem)
    d1.wait()
    d2.wait()
    d3.wait()

    def triple(t, c2):
      acc = jnp.zeros((16,), jnp.float32)
      for j in range(8):
        sl = pl.ds(16 * j, 16)
        v = rowsA[t, sl] + rowsB[t, sl] + rowsC[t, sl]
        acc = acc + wregs[j] * jnp.maximum(v, 0.0)
      o16[t] = acc
      return c2

    lax.fori_loop(0, 128, triple, 0)
    pltpu.sync_copy(o16, out_hbm.at[pl.ds((slab0 + k) * 128, 128)])
    return carry

  lax.fori_loop(0, TCH, chunk, 0)


def _decoder_sc(PA, PB, PC, place2d, srcn2d, dstn2d, step2d, wd2vec):
  return pl.kernel(
      _decoder_body,
      out_type=jax.ShapeDtypeStruct((T_PAD, 16), jnp.float32),
      mesh=_mesh,
      scratch_types=[
          pltpu.VMEM((TCH, 128), jnp.int32),
          pltpu.VMEM((TCH, 128), jnp.int32),
          pltpu.VMEM((TCH, 128), jnp.int32),
          pltpu.VMEM((TCH, 128), jnp.int32),
          pltpu.VMEM((128,), jnp.int32),
          pltpu.VMEM((128,), jnp.int32),
          pltpu.VMEM((128,), jnp.int32),
          pltpu.VMEM((128, D), jnp.float32),
          pltpu.VMEM((128, D), jnp.float32),
          pltpu.VMEM((128, D), jnp.float32),
          pltpu.VMEM((128, 16), jnp.float32),
          pltpu.VMEM((128,), jnp.float32),
          pltpu.SemaphoreType.DMA,
      ],
  )(PA, PB, PC, place2d, srcn2d, dstn2d, step2d, wd2vec)


def _logsig_body(x_ref, b_ref, o_ref):
  s = jnp.sum(x_ref[...], axis=1, keepdims=True) + b_ref[0, 0]
  o_ref[...] = jnp.broadcast_to(jax.nn.log_sigmoid(s), o_ref.shape)


def _logsig_tc(out16, bd2b):
  blk = 1024
  return pl.pallas_call(
      _logsig_body,
      grid=(T_PAD // blk,),
      in_specs=[
          pl.BlockSpec((blk, 16), lambda i: (i, 0)),
          pl.BlockSpec((1, 16), lambda i: (0, 0)),
      ],
      out_specs=pl.BlockSpec((blk, D), lambda i: (i, 0)),
      out_shape=jax.ShapeDtypeStruct((T_PAD, D), jnp.float32),
  )(out16, bd2b)


def _pred_body(lp_hbm, pl_hbm, z128_hbm, out_hbm, pslab, pidx, lpbuf,
               pred_sh):
  core = lax.axis_index("c")
  sub = lax.axis_index("s")
  wid = sub * NC + core
  r0 = sub * RSUB
  slab0 = wid * TCH
  pltpu.sync_copy(z128_hbm.at[pl.ds(r0, RSUB)], pred_sh.at[pl.ds(r0, RSUB)])
  pltpu.sync_copy(pl_hbm.at[pl.ds(slab0, TCH)], pslab)
  plsc.subcore_barrier()

  def chunk(k, carry):
    for q in range(8):
      sl = pl.ds(16 * q, 16)
      pidx[sl] = pslab[k, sl]
    pltpu.sync_copy(lp_hbm.at[pl.ds((slab0 + k) * 128, 128)], lpbuf)
    pltpu.sync_copy(lpbuf, pred_sh.at[pidx], add=True)
    return carry

  lax.fori_loop(0, TCH, chunk, 0)
  plsc.subcore_barrier()
  pltpu.sync_copy(pred_sh.at[pl.ds(r0, RSUB)],
                  out_hbm.at[core, pl.ds(r0, RSUB)])


def _pred_sc(lp128, place2d, zacc):
  return pl.kernel(
      _pred_body,
      out_type=jax.ShapeDtypeStruct((NC, NPAD, D), jnp.float32),
      mesh=_mesh,
      scratch_types=[
          pltpu.VMEM((TCH, 128), jnp.int32),
          pltpu.VMEM((128,), jnp.int32),
          pltpu.VMEM((128, D), jnp.float32),
          pltpu.VMEM_SHARED((NPAD, D), jnp.float32),
      ],
  )(lp128, place2d, zacc)


def _combine_body(p_ref, o_ref):
  o_ref[...] = p_ref[0] + p_ref[1]


def _combine_tc(predparts):
  return pl.pallas_call(
      _combine_body,
      grid=(NBLK,),
      in_specs=[pl.BlockSpec((NC, RB, D), lambda i: (0, i, 0))],
      out_specs=pl.BlockSpec((RB, D), lambda i: (i, 0)),
      out_shape=jax.ShapeDtypeStruct((NPAD, D), jnp.float32),
  )(predparts)


def kernel(x, edge_index, triples, step_ids, W1, b1, W2, b2, Wd1, bd1,
           Wd2, bd2):
  i32 = jnp.int32
  src = edge_index[0].astype(i32)
  dst = edge_index[1].astype(i32)
  src2d = jnp.concatenate(
      [src, jnp.zeros((E_PAD - E,), i32)]).reshape(-1, 128)
  dst2d = jnp.concatenate(
      [dst, jnp.full((E_PAD - E,), SINK, i32)]).reshape(-1, 128)

  pad_t = lambda a, fill: jnp.concatenate(
      [a.astype(i32), jnp.full((T_PAD - T,), fill, i32)]).reshape(-1, 128)
  place2d = pad_t(triples[:, 0], SINK)
  srcn2d = pad_t(triples[:, 1], 0)
  dstn2d = pad_t(triples[:, 2], 0)
  step2d = pad_t(step_ids, 0)

  zacc = jnp.zeros((NPAD, D), jnp.float32)
  ones128 = jnp.ones((128, D), jnp.float32)

  A, Bm, C = Wd1[0:D], Wd1[D:2 * D], Wd1[2 * D:3 * D]

  degparts = _deg_sc(dst2d, ones128, zacc)
  parts = _edge_agg(x, src2d, dst2d)
  h, pa, pb, pc = _layer_tc(parts, degparts, W1, b1, A, Bm, C, bd1)
  pas, pbs, pcs = [pa], [pb], [pc]
  for _ in range(S - 1):
    parts = _edge_agg(h, src2d, dst2d)
    h, pa, pb, pc = _layer_tc(parts, degparts, W2, b2, A, Bm, C, bd1)
    pas.append(pa)
    pbs.append(pb)
    pcs.append(pc)

  PA = jnp.concatenate(pas, axis=0)
  PB = jnp.concatenate(pbs, axis=0)
  PC = jnp.concatenate(pcs, axis=0)

  out16 = _decoder_sc(PA, PB, PC, place2d, srcn2d, dstn2d, step2d, Wd2[:, 0])
  lp128 = _logsig_tc(out16, jnp.broadcast_to(bd2.reshape(1, 1), (1, 16)))
  predparts = _pred_sc(lp128, place2d, zacc)
  pred = _combine_tc(predparts)
  return pred[:N, 0:1]


# speedup vs baseline: 2.8129x; 1.0029x over previous
import jax
import jax.numpy as jnp
from jax import lax
from jax.experimental import pallas as pl
from jax.experimental.pallas import tpu as pltpu
from jax.experimental.pallas import tpu_sc as plsc

N, D, E, T, S = 10000, 128, 320000, 100000, 4
NC, NS = 2, 16
NW = NC * NS
SINK = N
NPAD = 10240
RB = 512
NBLK = NPAD // RB
RSUB = NPAD // NS

EC = 10
EKI = 8
E_PAD = NW * EC * EKI * 128

TCH = 32
T_PAD = NW * TCH * 128

_mesh = plsc.VectorSubcoreMesh(
    core_axis_name="c", subcore_axis_name="s", num_cores=NC, num_subcores=NS)


def _edge_agg_body(h_hbm, src_hbm, dst_hbm, parts_out,
                   sidx, didx, didx1, rows, sem0, sem1, acc_sh):
  sems = (sem0, sem1)
  core = lax.axis_index("c")
  sub = lax.axis_index("s")
  wid = sub * NC + core
  r0 = sub * RSUB

  zv = jnp.zeros((16,), jnp.float32)

  def zrow(i, carry):
    for j in range(D // 16):
      rows[i, pl.ds(16 * j, 16)] = zv
    return carry

  lax.fori_loop(0, 128, zrow, 0)
  for m in range(RSUB // 128):
    pltpu.sync_copy(rows.at[pl.ds(0, 128)],
                    acc_sh.at[pl.ds(r0 + m * 128, 128)])
  plsc.subcore_barrier()

  def chunk(k, carry):
    base = (wid * EC + k) * EKI
    pltpu.sync_copy(src_hbm.at[pl.ds(base, EKI)], sidx)
    pltpu.sync_copy(dst_hbm.at[pl.ds(base, EKI)], didx)
    descs = [None, None]
    descs[0] = pltpu.async_copy(h_hbm.at[sidx.at[0]],
                                rows.at[pl.ds(0, 128)], sems[0])
    for j in range(EKI):
      if j + 1 < EKI:
        slot = (j + 1) % 2
        descs[slot] = pltpu.async_copy(h_hbm.at[sidx.at[j + 1]],
                                       rows.at[pl.ds(slot * 128, 128)],
                                       sems[slot])
      for q in range(8):
        sl = pl.ds(16 * q, 16)
        didx1[sl] = didx[j, sl]
      descs[j % 2].wait()
      pltpu.sync_copy(rows.at[pl.ds((j % 2) * 128, 128)],
                      acc_sh.at[didx1], add=True)
    return carry

  lax.fori_loop(0, EC, chunk, 0)
  plsc.subcore_barrier()
  for m in range(RSUB // 128):
    pltpu.sync_copy(acc_sh.at[pl.ds(r0 + m * 128, 128)],
                    rows.at[pl.ds(0, 128)])
    pltpu.sync_copy(rows.at[pl.ds(0, 128)],
                    parts_out.at[core, pl.ds(r0 + m * 128, 128)])


def _edge_agg(h, src2d, dst2d):
  return pl.kernel(
      _edge_agg_body,
      out_type=jax.ShapeDtypeStruct((NC, NPAD, D), jnp.float32),
      mesh=_mesh,
      scratch_types=[
          pltpu.VMEM((EKI, 128), jnp.int32),
          pltpu.VMEM((EKI, 128), jnp.int32),
          pltpu.VMEM((128,), jnp.int32),
          pltpu.VMEM((256, D), jnp.float32),
          pltpu.SemaphoreType.DMA,
          pltpu.SemaphoreType.DMA,
          pltpu.VMEM_SHARED((NPAD, D), jnp.float32),
      ],
  )(h, src2d, dst2d)


def _deg_body(dst_hbm, ones_hbm, z128_hbm, deg_out, didx, didx1, ones_v,
              deg_sh):
  core = lax.axis_index("c")
  sub = lax.axis_index("s")
  wid = sub * NC + core
  r0 = sub * RSUB
  pltpu.sync_copy(z128_hbm.at[pl.ds(r0, RSUB)], deg_sh.at[pl.ds(r0, RSUB)])
  pltpu.sync_copy(ones_hbm, ones_v)
  plsc.subcore_barrier()

  def chunk(k, carry):
    base = (wid * EC + k) * EKI
    pltpu.sync_copy(dst_hbm.at[pl.ds(base, EKI)], didx)
    for j in range(EKI):
      for q in range(8):
        sl = pl.ds(16 * q, 16)
        didx1[sl] = didx[j, sl]
      pltpu.sync_copy(ones_v, deg_sh.at[didx1], add=True)
    return carry

  lax.fori_loop(0, EC, chunk, 0)
  plsc.subcore_barrier()
  pltpu.sync_copy(deg_sh.at[pl.ds(r0, RSUB)],
                  deg_out.at[core, pl.ds(r0, RSUB)])


def _deg_sc(dst2d, ones128, zacc):
  return pl.kernel(
      _deg_body,
      out_type=jax.ShapeDtypeStruct((NC, NPAD, D), jnp.float32),
      mesh=_mesh,
      scratch_types=[
          pltpu.VMEM((EKI, 128), jnp.int32),
          pltpu.VMEM((128,), jnp.int32),
          pltpu.VMEM((128, D), jnp.float32),
          pltpu.VMEM_SHARED((NPAD, D), jnp.float32),
      ],
  )(dst2d, ones128, zacc)


def _layer_body(p_ref, dg_ref, w_ref, b_ref, a_ref, bm_ref, c_ref, bd_ref,
                h_ref, pa_ref, pb_ref, pc_ref):
  p = p_ref[0] + p_ref[1]
  dg = dg_ref[0, :, 0:1] + dg_ref[1, :, 0:1]
  agg = p * (1.0 / jnp.maximum(dg, 1.0))
  h = jnp.maximum(
      jnp.dot(agg, w_ref[...], preferred_element_type=jnp.float32) + b_ref[0],
      0.0)
  h_ref[...] = h
  pa_ref[...] = jnp.dot(h, a_ref[...],
                        preferred_element_type=jnp.float32) + bd_ref[0]
  pb_ref[...] = jnp.dot(h, bm_ref[...], preferred_element_type=jnp.float32)
  pc_ref[...] = jnp.dot(h, c_ref[...], preferred_element_type=jnp.float32)


def _layer_tc(parts, degparts, W, b, A, Bm, C, bd1):
  full = lambda shape: pl.BlockSpec(shape, lambda i: (0,) * len(shape))
  return pl.pallas_call(
      _layer_body,
      grid=(NBLK,),
      in_specs=[
          pl.BlockSpec((NC, RB, D), lambda i: (0, i, 0)),
          pl.BlockSpec((NC, RB, D), lambda i: (0, i, 0)),
          full((D, D)), full((1, D)),
          full((D, D)), full((D, D)), full((D, D)), full((1, D)),
      ],
      out_specs=[pl.BlockSpec((RB, D), lambda i: (i, 0))] * 4,
      out_shape=[jax.ShapeDtypeStruct((NPAD, D), jnp.float32)] * 4,
  )(parts, degparts, W, b.reshape(1, D), A, Bm, C, bd1.reshape(1, D))


def _decoder_body(pa_hbm, pb_hbm, pc_hbm, w1_hbm, w2_hbm,
                  wd2_hbm, out_hbm,
                  w1slab, w2slab,
                  ia0, ib0, ic0, ia1, ib1, ic1,
                  rA0, rB0, rC0, rA1, rB1, rC1, o16, wd2v, sem0, sem1):
  core = lax.axis_index("c")
  sub = lax.axis_index("s")
  wid = sub * NC + core
  slab0 = wid * TCH
  pltpu.sync_copy(wd2_hbm, wd2v)
  pltpu.sync_copy(w1_hbm.at[pl.ds(slab0, TCH)], w1slab)
  pltpu.sync_copy(w2_hbm.at[pl.ds(slab0, TCH)], w2slab)
  wregs = [wd2v[pl.ds(16 * j, 16)] for j in range(8)]

  slots = ((ia0, ib0, ic0, rA0, rB0, rC0, sem0),
           (ia1, ib1, ic1, rA1, rB1, rC1, sem1))

  def idx_into(k, slot):
    ia_, ib_, ic_ = slot[0], slot[1], slot[2]
    for j in range(8):
      sl = pl.ds(16 * j, 16)
      w1 = w1slab[k, sl]
      w2 = w2slab[k, sl]
      base = (w2 >> 16) * NPAD
      ia_[sl] = base + (w1 & 0xFFFF)
      ib_[sl] = base + (w1 >> 16)
      ic_[sl] = base + (w2 & 0xFFFF)

  def fire3(slot):
    ia_, ib_, ic_, rA_, rB_, rC_, sem = slot
    pltpu.async_copy(pa_hbm.at[ia_], rA_, sem)
    pltpu.async_copy(pb_hbm.at[ib_], rB_, sem)
    pltpu.async_copy(pc_hbm.at[ic_], rC_, sem)

  def drain3(slot):
    ia_, ib_, ic_, rA_, rB_, rC_, sem = slot
    pltpu.make_async_copy(pa_hbm.at[ia_], rA_, sem).wait()
    pltpu.make_async_copy(pb_hbm.at[ib_], rB_, sem).wait()
    pltpu.make_async_copy(pc_hbm.at[ic_], rC_, sem).wait()

  def compute_out(k, slot):
    rA_, rB_, rC_ = slot[3], slot[4], slot[5]

    @plsc.parallel_loop(0, 128, unroll=4)
    def triple(t):
      acc = jnp.zeros((16,), jnp.float32)
      for j in range(8):
        sl = pl.ds(16 * j, 16)
        v = rA_[t, sl] + rB_[t, sl] + rC_[t, sl]
        acc = acc + wregs[j] * jnp.maximum(v, 0.0)
      o16[t] = acc

    pltpu.sync_copy(o16, out_hbm.at[pl.ds((slab0 + k) * 128, 128)])

  idx_into(0, slots[0])
  fire3(slots[0])

  def pair(p, carry):
    k0 = 2 * p
    idx_into(k0 + 1, slots[1])
    fire3(slots[1])
    drain3(slots[0])
    compute_out(k0, slots[0])

    @pl.when(p < TCH // 2 - 1)
    def _():
      idx_into(k0 + 2, slots[0])
      fire3(slots[0])

    drain3(slots[1])
    compute_out(k0 + 1, slots[1])
    return carry

  lax.fori_loop(0, TCH // 2, pair, 0)


def _decoder_sc(PA, PB, PC, w1_2d, w2_2d, wd2vec):
  idx = lambda: pltpu.VMEM((128,), jnp.int32)
  rows = lambda: pltpu.VMEM((128, D), jnp.float32)
  return pl.kernel(
      _decoder_body,
      out_type=jax.ShapeDtypeStruct((T_PAD, 16), jnp.float32),
      mesh=_mesh,
      scratch_types=[
          pltpu.VMEM((TCH, 128), jnp.int32),
          pltpu.VMEM((TCH, 128), jnp.int32),
          idx(), idx(), idx(), idx(), idx(), idx(),
          rows(), rows(), rows(), rows(), rows(), rows(),
          pltpu.VMEM((128, 16), jnp.float32),
          pltpu.VMEM((128,), jnp.float32),
          pltpu.SemaphoreType.DMA,
          pltpu.SemaphoreType.DMA,
      ],
  )(PA, PB, PC, w1_2d, w2_2d, wd2vec)


def _logsig_body(x_ref, b_ref, o_ref):
  s = jnp.sum(x_ref[...], axis=1, keepdims=True) + b_ref[0, 0]
  o_ref[...] = jnp.broadcast_to(jax.nn.log_sigmoid(s), o_ref.shape)


def _logsig_tc(out16, bd2b):
  blk = 1024
  return pl.pallas_call(
      _logsig_body,
      grid=(T_PAD // blk,),
      in_specs=[
          pl.BlockSpec((blk, 16), lambda i: (i, 0)),
          pl.BlockSpec((1, 16), lambda i: (0, 0)),
      ],
      out_specs=pl.BlockSpec((blk, D), lambda i: (i, 0)),
      out_shape=jax.ShapeDtypeStruct((T_PAD, D), jnp.float32),
  )(out16, bd2b)


def _pred_body(lp_hbm, pl_hbm, z128_hbm, out_hbm, pslab, pidx, lpbuf,
               pred_sh):
  core = lax.axis_index("c")
  sub = lax.axis_index("s")
  wid = sub * NC + core
  r0 = sub * RSUB
  slab0 = wid * TCH
  pltpu.sync_copy(z128_hbm.at[pl.ds(r0, RSUB)], pred_sh.at[pl.ds(r0, RSUB)])
  pltpu.sync_copy(pl_hbm.at[pl.ds(slab0, TCH)], pslab)
  plsc.subcore_barrier()

  def chunk(k, carry):
    for q in range(8):
      sl = pl.ds(16 * q, 16)
      pidx[sl] = pslab[k, sl]
    pltpu.sync_copy(lp_hbm.at[pl.ds((slab0 + k) * 128, 128)], lpbuf)
    pltpu.sync_copy(lpbuf, pred_sh.at[pidx], add=True)
    return carry

  lax.fori_loop(0, TCH, chunk, 0)
  plsc.subcore_barrier()
  pltpu.sync_copy(pred_sh.at[pl.ds(r0, RSUB)],
                  out_hbm.at[core, pl.ds(r0, RSUB)])


def _pred_sc(lp128, place2d, zacc):
  return pl.kernel(
      _pred_body,
      out_type=jax.ShapeDtypeStruct((NC, NPAD, D), jnp.float32),
      mesh=_mesh,
      scratch_types=[
          pltpu.VMEM((TCH, 128), jnp.int32),
          pltpu.VMEM((128,), jnp.int32),
          pltpu.VMEM((128, D), jnp.float32),
          pltpu.VMEM_SHARED((NPAD, D), jnp.float32),
      ],
  )(lp128, place2d, zacc)


def _combine_body(p_ref, o_ref):
  o_ref[...] = p_ref[0] + p_ref[1]


def _combine_tc(predparts):
  return pl.pallas_call(
      _combine_body,
      grid=(NBLK,),
      in_specs=[pl.BlockSpec((NC, RB, D), lambda i: (0, i, 0))],
      out_specs=pl.BlockSpec((RB, D), lambda i: (i, 0)),
      out_shape=jax.ShapeDtypeStruct((NPAD, D), jnp.float32),
  )(predparts)


def kernel(x, edge_index, triples, step_ids, W1, b1, W2, b2, Wd1, bd1,
           Wd2, bd2):
  i32 = jnp.int32
  src = edge_index[0].astype(i32)
  dst = edge_index[1].astype(i32)
  src2d = jnp.concatenate(
      [src, jnp.zeros((E_PAD - E,), i32)]).reshape(-1, 128)
  dst2d = jnp.concatenate(
      [dst, jnp.full((E_PAD - E,), SINK, i32)]).reshape(-1, 128)

  pad_t = lambda a, fill: jnp.concatenate(
      [a.astype(i32), jnp.full((T_PAD - T,), fill, i32)]).reshape(-1, 128)
  place2d = pad_t(triples[:, 0], SINK)
  w1_2d = pad_t(triples[:, 0] + (triples[:, 1] << 16), SINK)
  w2_2d = pad_t(triples[:, 2] + (step_ids << 16), 0)

  zacc = jnp.zeros((NPAD, D), jnp.float32)
  ones128 = jnp.ones((128, D), jnp.float32)

  A, Bm, C = Wd1[0:D], Wd1[D:2 * D], Wd1[2 * D:3 * D]

  degparts = _deg_sc(dst2d, ones128, zacc)
  parts = _edge_agg(x, src2d, dst2d)
  h, pa, pb, pc = _layer_tc(parts, degparts, W1, b1, A, Bm, C, bd1)
  pas, pbs, pcs = [pa], [pb], [pc]
  for _ in range(S - 1):
    parts = _edge_agg(h, src2d, dst2d)
    h, pa, pb, pc = _layer_tc(parts, degparts, W2, b2, A, Bm, C, bd1)
    pas.append(pa)
    pbs.append(pb)
    pcs.append(pc)

  PA = jnp.concatenate(pas, axis=0)
  PB = jnp.concatenate(pbs, axis=0)
  PC = jnp.concatenate(pcs, axis=0)

  out16 = _decoder_sc(PA, PB, PC, w1_2d, w2_2d, Wd2[:, 0])
  lp128 = _logsig_tc(out16, jnp.broadcast_to(bd2.reshape(1, 1), (1, 16)))
  predparts = _pred_sc(lp128, place2d, zacc)
  pred = _combine_tc(predparts)
  return pred[:N, 0:1]
